# Initial kernel scaffold; baseline (speedup 1.0000x reference)
import jax
import jax.numpy as jnp
from jax.experimental import pallas as pl


def kernel(sent_vecs, concept_ids, node_type_ids, edge_index, edge_type, ee_W0, ee_b0, ee_g, ee_be, ee_W1, ee_b1, reg_W0, reg_b0, reg_g, reg_be, reg_W1, reg_b1, fc_W0, fc_b0, fc_g, fc_be, fc_W1, fc_b1):
    raise NotImplementedError("write your pallas kernel here")


# speedup vs baseline: 102.6838x; 102.6838x over previous
import functools

import jax
import jax.numpy as jnp
import numpy as np
from jax import lax
from jax.experimental import pallas as pl
from jax.experimental.pallas import tpu as pltpu
from jax.experimental.pallas import tpu_sc as plsc

NNT = 4

NC = 2
NS = 16
LANES = 16
ROW = 128
WR = 16


def _ln(x, g, b):
    m = x.mean(-1, keepdims=True)
    v = x.var(-1, keepdims=True)
    return (x - m) / jnp.sqrt(v + 1e-5) * g + b


def _tc_pre(feat, ee_W0p, ee_b0, ee_g, ee_be, ee_W1, ee_b1,
            sent_vecs, fc_W0, fc_b0, fc_g, fc_be, fc_W1, fc_b1):
    def body(feat_ref, w0_ref, b0_ref, g_ref, be_ref, w1_ref, b1_ref,
             sv_ref, fw0_ref, fb0_ref, fg_ref, fbe_ref, fw1_ref, fb1_ref,
             tab_ref, qc_ref):
        h = jnp.dot(feat_ref[...], w0_ref[...],
                    preferred_element_type=jnp.float32) + b0_ref[...]
        h = jax.nn.gelu(_ln(h, g_ref[...], be_ref[...]))
        tab_ref[...] = jax.nn.sigmoid(
            jnp.dot(h, w1_ref[...], preferred_element_type=jnp.float32)
            + b1_ref[...])
        q = jnp.dot(sv_ref[...], fw0_ref[...],
                    preferred_element_type=jnp.float32) + fb0_ref[...]
        q = jax.nn.gelu(_ln(q, fg_ref[...], fbe_ref[...]))
        qc_ref[...] = (jnp.dot(q, fw1_ref[...],
                               preferred_element_type=jnp.float32)
                       + fb1_ref[...])

    ncomb = feat.shape[0]
    b = sent_vecs.shape[0]
    return pl.pallas_call(
        body,
        out_shape=(jax.ShapeDtypeStruct((ncomb, 1), jnp.float32),
                   jax.ShapeDtypeStruct((b, 1), jnp.float32)),
    )(feat, ee_W0p, ee_b0, ee_g, ee_be, ee_W1, ee_b1,
      sent_vecs, fc_W0, fc_b0, fc_g, fc_be, fc_W1, fc_b1)


def _tc_post(p0, p1, qc, reg_W0, reg_b0, reg_g, reg_be, reg_W1, reg_b1):
    def body(p0_ref, p1_ref, qc_ref, rw0_ref, rb0_ref, rg_ref, rbe_ref,
             rw1_ref, rb1_ref, out_ref):
        x = p0_ref[...] + p1_ref[...]
        h = x * rw0_ref[...] + rb0_ref[...]
        h = jax.nn.gelu(_ln(h, rg_ref[...], rbe_ref[...]))
        gm = (jnp.dot(h, rw1_ref[...], preferred_element_type=jnp.float32)
              + rb1_ref[...])
        out_ref[...] = gm + qc_ref[...]

    b = p0.shape[0]
    return pl.pallas_call(
        body,
        out_shape=jax.ShapeDtypeStruct((b, 1), jnp.float32),
    )(p0, p1, qc, reg_W0, reg_b0, reg_g, reg_be, reg_W1, reg_b1)


def _mesh():
    return plsc.VectorSubcoreMesh(core_axis_name="c", subcore_axis_name="s",
                                  num_cores=NC, num_subcores=NS)


def _zero_stripe(zbuf, acc_sp, sid, stripe):
    def zf(i, c):
        zbuf[pl.ds(i * LANES, LANES)] = jnp.zeros((LANES,), jnp.float32)
        return c
    lax.fori_loop(0, stripe // LANES, zf, 0)
    pltpu.sync_copy(zbuf, acc_sp.at[pl.ds(sid * stripe, stripe)])


def _scatter_window(m_v, dst_v, acc_sp, sem):
    descs = [pltpu.async_copy(m_v.at[j], acc_sp.at[dst_v.at[j]], sem,
                              add=True) for j in range(WR)]
    for d in descs:
        d.wait()


def _sc_round1(et2d, src2d, dst2d, nt_pad, table, np_pad, rt):
    r_all = et2d.shape[0]
    stripe = np_pad // NS
    nwin = rt // WR
    ncomb = table.shape[0]

    @functools.partial(
        pl.kernel,
        out_type=(jax.ShapeDtypeStruct((r_all, ROW), jnp.float32),
                  jax.ShapeDtypeStruct((NC * np_pad,), jnp.float32)),
        mesh=_mesh(),
        compiler_params=pltpu.CompilerParams(needs_layout_passes=False),
        scratch_types=[
            pltpu.VMEM((np_pad,), jnp.int32),
            pltpu.VMEM((ncomb,), jnp.float32),
            pltpu.VMEM((WR, ROW), jnp.int32),
            pltpu.VMEM((WR, ROW), jnp.int32),
            pltpu.VMEM((WR, ROW), jnp.int32),
            pltpu.VMEM((WR, ROW), jnp.float32),
            pltpu.VMEM((stripe,), jnp.float32),
            pltpu.VMEM_SHARED((np_pad,), jnp.float32),
            pltpu.SemaphoreType.DMA,
        ],
    )
    def k(et_hbm, src_hbm, dst_hbm, nt_hbm, tab_hbm, e_out, p_out,
          nt_v, tab_v, et_v, src_v, dst_v, m_v, zbuf, acc_sp, sem):
        cid = lax.axis_index("c")
        sid = lax.axis_index("s")
        wid = cid * NS + sid
        pltpu.sync_copy(nt_hbm, nt_v)
        pltpu.sync_copy(tab_hbm, tab_v)
        _zero_stripe(zbuf, acc_sp, sid, stripe)
        plsc.subcore_barrier()
        base = wid * rt
        for w in range(nwin):
            r0 = base + w * WR
            pltpu.sync_copy(et_hbm.at[pl.ds(r0, WR)], et_v)
            pltpu.sync_copy(src_hbm.at[pl.ds(r0, WR)], src_v)
            pltpu.sync_copy(dst_hbm.at[pl.ds(r0, WR)], dst_v)

            def cf(r, c):
                for g in range(ROW // LANES):
                    sl = pl.ds(g * LANES, LANES)
                    et = et_v[r, sl]
                    s = src_v[r, sl]
                    d = dst_v[r, sl]
                    ht = plsc.load_gather(nt_v, [s])
                    tt = plsc.load_gather(nt_v, [d])
                    idx = et * (NNT * NNT) + ht * NNT + tt
                    m_v[r, sl] = plsc.load_gather(tab_v, [idx])
                return c
            lax.fori_loop(0, WR, cf, 0)
            pltpu.sync_copy(m_v, e_out.at[pl.ds(r0, WR)])
            _scatter_window(m_v, dst_v, acc_sp, sem)
        plsc.subcore_barrier()
        sl = pl.ds(sid * stripe, stripe)
        pltpu.sync_copy(acc_sp.at[sl], zbuf)
        pltpu.sync_copy(zbuf,
                        p_out.at[pl.ds(cid * np_pad + sid * stripe, stripe)])

    return k(et2d, src2d, dst2d, nt_pad, table)


def _sc_round(src2d, dst2d, e2d, p_in, np_pad, rt):
    stripe = np_pad // NS
    nwin = rt // WR
    chunk = np_pad // 8

    @functools.partial(
        pl.kernel,
        out_type=jax.ShapeDtypeStruct((NC * np_pad,), jnp.float32),
        mesh=_mesh(),
        compiler_params=pltpu.CompilerParams(needs_layout_passes=False),
        scratch_types=[
            pltpu.VMEM((np_pad,), jnp.float32),
            pltpu.VMEM((chunk,), jnp.float32),
            pltpu.VMEM((chunk,), jnp.float32),
            pltpu.VMEM((WR, ROW), jnp.int32),
            pltpu.VMEM((WR, ROW), jnp.int32),
            pltpu.VMEM((WR, ROW), jnp.float32),
            pltpu.VMEM((WR, ROW), jnp.float32),
            pltpu.VMEM((stripe,), jnp.float32),
            pltpu.VMEM_SHARED((np_pad,), jnp.float32),
            pltpu.SemaphoreType.DMA,
        ],
    )
    def k(src_hbm, dst_hbm, e_hbm, p_hbm, p_out,
          a_v, t0, t1, src_v, dst_v, e_v, m_v, zbuf, acc_sp, sem):
        cid = lax.axis_index("c")
        sid = lax.axis_index("s")
        wid = cid * NS + sid
        for ch in range(8):
            off = ch * chunk
            pltpu.sync_copy(p_hbm.at[pl.ds(off, chunk)], t0)
            pltpu.sync_copy(p_hbm.at[pl.ds(np_pad + off, chunk)], t1)

            def af(i, c):
                sl = pl.ds(i * LANES, LANES)
                a_v[pl.ds(off + i * LANES, LANES)] = t0[sl] + t1[sl]
                return c
            lax.fori_loop(0, chunk // LANES, af, 0)
        _zero_stripe(zbuf, acc_sp, sid, stripe)
        plsc.subcore_barrier()
        base = wid * rt
        for w in range(nwin):
            r0 = base + w * WR
            pltpu.sync_copy(src_hbm.at[pl.ds(r0, WR)], src_v)
            pltpu.sync_copy(dst_hbm.at[pl.ds(r0, WR)], dst_v)
            pltpu.sync_copy(e_hbm.at[pl.ds(r0, WR)], e_v)

            def cf(r, c):
                for g in range(ROW // LANES):
                    sl = pl.ds(g * LANES, LANES)
                    s = src_v[r, sl]
                    m_v[r, sl] = plsc.load_gather(a_v, [s]) + e_v[r, sl]
                return c
            lax.fori_loop(0, WR, cf, 0)
            _scatter_window(m_v, dst_v, acc_sp, sem)
        plsc.subcore_barrier()
        sl = pl.ds(sid * stripe, stripe)
        pltpu.sync_copy(acc_sp.at[sl], zbuf)
        pltpu.sync_copy(zbuf,
                        p_out.at[pl.ds(cid * np_pad + sid * stripe, stripe)])

    return k(src2d, dst2d, e2d, p_in)


def kernel(sent_vecs, concept_ids, node_type_ids, edge_index, edge_type,
           ee_W0, ee_b0, ee_g, ee_be, ee_W1, ee_b1,
           reg_W0, reg_b0, reg_g, reg_be, reg_W1, reg_b1,
           fc_W0, fc_b0, fc_g, fc_be, fc_W1, fc_b1):
    b, nn = node_type_ids.shape
    n = b * nn
    e_cnt = edge_type.shape[0]
    d = ee_W0.shape[0]
    net = d - 2 * NNT
    k_rounds = 4

    np_pad = -((n + 1) // -ROW) * ROW
    tiles = NC * NS
    rt = -(e_cnt // -(tiles * WR * ROW)) * WR * ROW // ROW
    e_pad = tiles * rt * ROW
    r_all = e_pad // ROW

    src = edge_index[0].astype(jnp.int32)
    dst = edge_index[1].astype(jnp.int32)
    et = edge_type.astype(jnp.int32)
    pad = e_pad - e_cnt
    src2d = jnp.concatenate([src, jnp.zeros((pad,), jnp.int32)]).reshape(r_all, ROW)
    dst2d = jnp.concatenate([dst, jnp.full((pad,), n, jnp.int32)]).reshape(r_all, ROW)
    et2d = jnp.concatenate([et, jnp.zeros((pad,), jnp.int32)]).reshape(r_all, ROW)
    nt_pad = jnp.concatenate([node_type_ids.reshape(-1).astype(jnp.int32),
                              jnp.zeros((np_pad - n,), jnp.int32)])

    ncomb = net * NNT * NNT
    ii = np.arange(ncomb)
    feat = np.zeros((ncomb, 128), np.float32)
    feat[ii, ii // (NNT * NNT)] = 1.0
    feat[ii, net + (ii // NNT) % NNT] = 1.0
    feat[ii, net + NNT + ii % NNT] = 1.0
    ee_W0p = jnp.zeros((128, ee_W0.shape[1]), jnp.float32).at[:d].set(ee_W0)

    table, qc = _tc_pre(jnp.asarray(feat), ee_W0p, ee_b0, ee_g, ee_be,
                        ee_W1, ee_b1, sent_vecs,
                        fc_W0, fc_b0, fc_g, fc_be, fc_W1, fc_b1)
    table = table.reshape(ncomb)

    e2d, p = _sc_round1(et2d, src2d, dst2d, nt_pad, table, np_pad, rt)
    for _ in range(k_rounds - 1):
        p = _sc_round(src2d, dst2d, e2d, p, np_pad, rt)

    roots = p.reshape(NC, np_pad)[:, :n].reshape(NC, b, nn)[:, :, 0:1]
    return _tc_post(roots[0], roots[1], qc,
                    reg_W0, reg_b0, reg_g, reg_be, reg_W1, reg_b1)


# speedup vs baseline: 128.5636x; 1.2520x over previous
import functools

import jax
import jax.numpy as jnp
import numpy as np
from jax import lax
from jax.experimental import pallas as pl
from jax.experimental.pallas import tpu as pltpu
from jax.experimental.pallas import tpu_sc as plsc

NNT = 4

NC = 2
NS = 16
TILES = NC * NS
LANES = 16
ROW = 128
WR = 8
NBUF = 2


def _ln(x, g, b):
    m = x.mean(-1, keepdims=True)
    v = x.var(-1, keepdims=True)
    return (x - m) / jnp.sqrt(v + 1e-5) * g + b


def _tc_pre(feat, ee_W0p, ee_b0, ee_g, ee_be, ee_W1, ee_b1,
            sent_vecs, fc_W0, fc_b0, fc_g, fc_be, fc_W1, fc_b1):
    def body(feat_ref, w0_ref, b0_ref, g_ref, be_ref, w1_ref, b1_ref,
             sv_ref, fw0_ref, fb0_ref, fg_ref, fbe_ref, fw1_ref, fb1_ref,
             tab_ref, qc_ref):
        h = jnp.dot(feat_ref[...], w0_ref[...],
                    preferred_element_type=jnp.float32) + b0_ref[...]
        h = jax.nn.gelu(_ln(h, g_ref[...], be_ref[...]))
        tab_ref[...] = jax.nn.sigmoid(
            jnp.dot(h, w1_ref[...], preferred_element_type=jnp.float32)
            + b1_ref[...])
        q = jnp.dot(sv_ref[...], fw0_ref[...],
                    preferred_element_type=jnp.float32) + fb0_ref[...]
        q = jax.nn.gelu(_ln(q, fg_ref[...], fbe_ref[...]))
        qc_ref[...] = (jnp.dot(q, fw1_ref[...],
                               preferred_element_type=jnp.float32)
                       + fb1_ref[...])

    ncomb = feat.shape[0]
    b = sent_vecs.shape[0]
    return pl.pallas_call(
        body,
        out_shape=(jax.ShapeDtypeStruct((ncomb, 1), jnp.float32),
                   jax.ShapeDtypeStruct((b, 1), jnp.float32)),
    )(feat, ee_W0p, ee_b0, ee_g, ee_be, ee_W1, ee_b1,
      sent_vecs, fc_W0, fc_b0, fc_g, fc_be, fc_W1, fc_b1)


def _tc_reduce(p_flat, np_pad):
    def body(p_ref, o_ref):
        o_ref[...] = jnp.sum(p_ref[...], axis=0)

    return pl.pallas_call(
        body,
        out_shape=jax.ShapeDtypeStruct((np_pad,), jnp.float32),
    )(p_flat.reshape(TILES, np_pad))


def _tc_post(proots, qc, reg_W0, reg_b0, reg_g, reg_be, reg_W1, reg_b1):
    def body(pr_ref, qc_ref, rw0_ref, rb0_ref, rg_ref, rbe_ref,
             rw1_ref, rb1_ref, out_ref):
        x = jnp.sum(pr_ref[...], axis=0)[:, None]
        h = x * rw0_ref[...] + rb0_ref[...]
        h = jax.nn.gelu(_ln(h, rg_ref[...], rbe_ref[...]))
        gm = (jnp.dot(h, rw1_ref[...], preferred_element_type=jnp.float32)
              + rb1_ref[...])
        out_ref[...] = gm + qc_ref[...]

    b = proots.shape[1]
    return pl.pallas_call(
        body,
        out_shape=jax.ShapeDtypeStruct((b, 1), jnp.float32),
    )(proots, qc, reg_W0, reg_b0, reg_g, reg_be, reg_W1, reg_b1)


def _mesh():
    return plsc.VectorSubcoreMesh(core_axis_name="c", subcore_axis_name="s",
                                  num_cores=NC, num_subcores=NS)


def _zero_acc(acc_v, np_pad):
    def zf(i, c):
        acc_v[pl.ds(i * LANES, LANES)] = jnp.zeros((LANES,), jnp.float32)
        return c
    lax.fori_loop(0, np_pad // LANES, zf, 0)


def _sc_round1(ed3, nt_pad, table, np_pad, rt):
    r_all = ed3.shape[0] // 3
    nwin = rt // WR
    assert nwin % NBUF == 0
    ncomb = table.shape[0]

    @functools.partial(
        pl.kernel,
        out_type=(jax.ShapeDtypeStruct((r_all, ROW), jnp.float32),
                  jax.ShapeDtypeStruct((TILES * np_pad,), jnp.float32)),
        mesh=_mesh(),
        compiler_params=pltpu.CompilerParams(needs_layout_passes=False),
        scratch_types=[
            pltpu.VMEM((np_pad,), jnp.int32),
            pltpu.VMEM((np_pad,), jnp.float32),
            pltpu.VMEM((ncomb,), jnp.float32),
            pltpu.VMEM((NBUF * WR * 3, ROW), jnp.int32),
            pltpu.VMEM((NBUF * WR, ROW), jnp.float32),
            pltpu.SemaphoreType.DMA,
            pltpu.SemaphoreType.DMA, pltpu.SemaphoreType.DMA,
            pltpu.SemaphoreType.DMA, pltpu.SemaphoreType.DMA,
        ],
    )
    def k(ed_hbm, nt_hbm, tab_hbm, e_out, p_out,
          nt_v, acc_v, tab_v, ibuf, fbuf, stg, l0, l1, e0, e1):
        lsem = (l0, l1)
        esem = (e0, e1)
        cid = lax.axis_index("c")
        sid = lax.axis_index("s")
        wid = cid * NS + sid
        base = wid * rt
        d0 = pltpu.async_copy(nt_hbm, nt_v, stg)
        d1 = pltpu.async_copy(tab_hbm, tab_v, stg)
        pltpu.async_copy(ed_hbm.at[pl.ds(base * 3, WR * 3)],
                         ibuf.at[pl.ds(0, WR * 3)], lsem[0])
        _zero_acc(acc_v, np_pad)
        d0.wait()
        d1.wait()

        def wf(i, c):
            for k2 in range(NBUF):
                w = i * NBUF + k2
                nslot = (k2 + 1) % NBUF
                r0 = base + w * WR

                @pl.when(w >= 2)
                def _():
                    pltpu.make_async_copy(
                        fbuf.at[pl.ds(k2 * WR, WR)],
                        e_out.at[pl.ds(r0, WR)], esem[k2]).wait()

                @pl.when(w + 1 < nwin)
                def _():
                    pltpu.async_copy(
                        ed_hbm.at[pl.ds((r0 + WR) * 3, WR * 3)],
                        ibuf.at[pl.ds(nslot * WR * 3, WR * 3)], lsem[nslot])

                pltpu.make_async_copy(ed_hbm.at[pl.ds(r0 * 3, WR * 3)],
                                      ibuf.at[pl.ds(k2 * WR * 3, WR * 3)],
                                      lsem[k2]).wait()

                def cf(r, cc):
                    row = (k2 * WR + r) * 3
                    frow = k2 * WR + r
                    for g in range(ROW // LANES):
                        sl = pl.ds(g * LANES, LANES)
                        et = ibuf[row, sl]
                        s = ibuf[row + 1, sl]
                        d = ibuf[row + 2, sl]
                        ht = plsc.load_gather(nt_v, [s])
                        tt = plsc.load_gather(nt_v, [d])
                        idx = et * (NNT * NNT) + ht * NNT + tt
                        e = plsc.load_gather(tab_v, [idx])
                        fbuf[frow, sl] = e
                        plsc.addupdate_scatter(acc_v, [d], e)
                    return cc
                lax.fori_loop(0, WR, cf, 0)
                pltpu.async_copy(fbuf.at[pl.ds(k2 * WR, WR)],
                                 e_out.at[pl.ds(r0, WR)], esem[k2])
            return c
        lax.fori_loop(0, nwin // NBUF, wf, 0)
        for w in (nwin - 2, nwin - 1):
            pltpu.make_async_copy(fbuf.at[pl.ds((w % NBUF) * WR, WR)],
                                  e_out.at[pl.ds(base, WR)],
                                  esem[w % NBUF]).wait()
        pltpu.sync_copy(acc_v, p_out.at[pl.ds(wid * np_pad, np_pad)])

    return k(ed3, nt_pad, table)


def _sc_round(sd2, e2d, a_hbm, np_pad, rt):
    nwin = rt // WR
    assert nwin % NBUF == 0

    @functools.partial(
        pl.kernel,
        out_type=jax.ShapeDtypeStruct((TILES * np_pad,), jnp.float32),
        mesh=_mesh(),
        compiler_params=pltpu.CompilerParams(needs_layout_passes=False),
        scratch_types=[
            pltpu.VMEM((np_pad,), jnp.float32),
            pltpu.VMEM((np_pad,), jnp.float32),
            pltpu.VMEM((NBUF * WR * 2, ROW), jnp.int32),
            pltpu.VMEM((NBUF * WR, ROW), jnp.float32),
            pltpu.SemaphoreType.DMA,
            pltpu.SemaphoreType.DMA, pltpu.SemaphoreType.DMA,
        ],
    )
    def k(sd_hbm, e_hbm, a_in, p_out, a_v, acc_v, ibuf, fbuf, stg, l0, l1):
        lsem = (l0, l1)
        cid = lax.axis_index("c")
        sid = lax.axis_index("s")
        wid = cid * NS + sid
        base = wid * rt
        d0 = pltpu.async_copy(a_in, a_v, stg)
        pltpu.async_copy(sd_hbm.at[pl.ds(base * 2, WR * 2)],
                         ibuf.at[pl.ds(0, WR * 2)], lsem[0])
        pltpu.async_copy(e_hbm.at[pl.ds(base, WR)],
                         fbuf.at[pl.ds(0, WR)], lsem[0])
        _zero_acc(acc_v, np_pad)
        d0.wait()

        def wf(i, c):
            for k2 in range(NBUF):
                w = i * NBUF + k2
                nslot = (k2 + 1) % NBUF
                r0 = base + w * WR

                @pl.when(w + 1 < nwin)
                def _():
                    pltpu.async_copy(
                        sd_hbm.at[pl.ds((r0 + WR) * 2, WR * 2)],
                        ibuf.at[pl.ds(nslot * WR * 2, WR * 2)], lsem[nslot])
                    pltpu.async_copy(e_hbm.at[pl.ds(r0 + WR, WR)],
                                     fbuf.at[pl.ds(nslot * WR, WR)],
                                     lsem[nslot])

                pltpu.make_async_copy(sd_hbm.at[pl.ds(r0 * 2, WR * 2)],
                                      ibuf.at[pl.ds(k2 * WR * 2, WR * 2)],
                                      lsem[k2]).wait()
                pltpu.make_async_copy(e_hbm.at[pl.ds(r0, WR)],
                                      fbuf.at[pl.ds(k2 * WR, WR)],
                                      lsem[k2]).wait()

                def cf(r, cc):
                    row = (k2 * WR + r) * 2
                    frow = k2 * WR + r
                    for g in range(ROW // LANES):
                        sl = pl.ds(g * LANES, LANES)
                        s = ibuf[row, sl]
                        d = ibuf[row + 1, sl]
                        v = plsc.load_gather(a_v, [s]) + fbuf[frow, sl]
                        plsc.addupdate_scatter(acc_v, [d], v)
                    return cc
                lax.fori_loop(0, WR, cf, 0)
            return c
        lax.fori_loop(0, nwin // NBUF, wf, 0)
        pltpu.sync_copy(acc_v, p_out.at[pl.ds(wid * np_pad, np_pad)])

    return k(sd2, e2d, a_hbm)


def kernel(sent_vecs, concept_ids, node_type_ids, edge_index, edge_type,
           ee_W0, ee_b0, ee_g, ee_be, ee_W1, ee_b1,
           reg_W0, reg_b0, reg_g, reg_be, reg_W1, reg_b1,
           fc_W0, fc_b0, fc_g, fc_be, fc_W1, fc_b1):
    b, nn = node_type_ids.shape
    n = b * nn
    e_cnt = edge_type.shape[0]
    d = ee_W0.shape[0]
    net = d - 2 * NNT
    k_rounds = 4

    np_pad = -((n + 1) // -ROW) * ROW
    blk = WR * NBUF * ROW
    rt = -(e_cnt // -(TILES * blk)) * blk // ROW
    e_pad = TILES * rt * ROW
    r_all = e_pad // ROW

    src = edge_index[0].astype(jnp.int32)
    dst = edge_index[1].astype(jnp.int32)
    et = edge_type.astype(jnp.int32)
    pad = e_pad - e_cnt
    src2d = jnp.concatenate([src, jnp.zeros((pad,), jnp.int32)]).reshape(r_all, ROW)
    dst2d = jnp.concatenate([dst, jnp.full((pad,), n, jnp.int32)]).reshape(r_all, ROW)
    et2d = jnp.concatenate([et, jnp.zeros((pad,), jnp.int32)]).reshape(r_all, ROW)
    ed3 = jnp.stack([et2d, src2d, dst2d], axis=1).reshape(r_all * 3, ROW)
    sd2 = jnp.stack([src2d, dst2d], axis=1).reshape(r_all * 2, ROW)
    nt_pad = jnp.concatenate([node_type_ids.reshape(-1).astype(jnp.int32),
                              jnp.zeros((np_pad - n,), jnp.int32)])

    ncomb = net * NNT * NNT
    ii = np.arange(ncomb)
    feat = np.zeros((ncomb, 128), np.float32)
    feat[ii, ii // (NNT * NNT)] = 1.0
    feat[ii, net + (ii // NNT) % NNT] = 1.0
    feat[ii, net + NNT + ii % NNT] = 1.0
    ee_W0p = jnp.zeros((128, ee_W0.shape[1]), jnp.float32).at[:d].set(ee_W0)

    table, qc = _tc_pre(jnp.asarray(feat), ee_W0p, ee_b0, ee_g, ee_be,
                        ee_W1, ee_b1, sent_vecs,
                        fc_W0, fc_b0, fc_g, fc_be, fc_W1, fc_b1)
    table = table.reshape(ncomb)

    e2d, p = _sc_round1(ed3, nt_pad, table, np_pad, rt)
    for _ in range(k_rounds - 2):
        a = _tc_reduce(p, np_pad)
        p = _sc_round(sd2, e2d, a, np_pad, rt)
    a = _tc_reduce(p, np_pad)
    p = _sc_round(sd2, e2d, a, np_pad, rt)

    proots = (p.reshape(TILES, np_pad)[:, :n]
              .reshape(TILES, b, nn)[:, :, 0])
    return _tc_post(proots, qc,
                    reg_W0, reg_b0, reg_g, reg_be, reg_W1, reg_b1)
